# Initial kernel scaffold; baseline (speedup 1.0000x reference)
import jax
import jax.numpy as jnp
from jax.experimental import pallas as pl


def kernel(x, up_weights, down_weights):
    raise NotImplementedError("write your pallas kernel here")


# speedup vs baseline: 6.3967x; 6.3967x over previous
import jax
import jax.numpy as jnp
from jax.experimental import pallas as pl
from jax.experimental.pallas import tpu as pltpu

D = 2048
STRIDES = (2, 4, 8)
R = 256


def _body(w_ref, x_ref, o_ref):
    xb = x_ref[...]
    xt = xb.T
    out = xt
    for i, s in enumerate(STRIDES):
        n = D // s
        uw = w_ref[i]
        dw = w_ref[3 + i]
        pref = (xt[:n] * uw)[:, None, :]
        dil = jnp.concatenate(
            [pref, jnp.zeros((n, s - 1, R), jnp.float32)], axis=1
        ).reshape(D, R)
        pooled = xt.reshape(n, s, R).sum(axis=1) * dw
        row = jax.lax.broadcasted_iota(jnp.int32, (n, R), 0)
        pooled = jnp.where(row >= 1, pooled, 0.0)
        down = jnp.concatenate(
            [pooled, jnp.zeros((D - n, R), jnp.float32)], axis=0
        )
        out = out + dil + down
    o_ref[...] = out.T


def kernel(x, up_weights, down_weights):
    B, S, d = x.shape
    xf = x.reshape(B * S, d)
    w = jnp.concatenate([jax.nn.sigmoid(up_weights), jax.nn.sigmoid(down_weights)])
    out = pl.pallas_call(
        _body,
        grid=(B * S // R,),
        in_specs=[
            pl.BlockSpec(memory_space=pltpu.SMEM),
            pl.BlockSpec((R, D), lambda i: (i, 0)),
        ],
        out_specs=pl.BlockSpec((R, D), lambda i: (i, 0)),
        out_shape=jax.ShapeDtypeStruct((B * S, D), jnp.float32),
    )(w, xf)
    return out.reshape(B, S, d)
